# baseline (speedup 1.0000x reference)
import functools

import jax
import jax.numpy as jnp
from jax.experimental import pallas as pl
from jax.experimental.pallas import tpu as pltpu

_N = 10000
_C = 128
_H = 4
_R = 16
_G = 16
_M = 256
_F = _H * _C
_NB = 10000
_NSTEPS = _N // _NB


def _bipart_kernel(x_ref, batch_ref, wl_ref, blt_ref, xcb_ref, wr_ref, brt_ref,
                   att_ref, biast_ref, out_ref, acc_ref, xlself_ref,
                   corrf_ref):
    step = pl.program_id(0)

    x_blk = x_ref[...]
    xlt = jax.lax.dot_general(wl_ref[...], x_blk, (((1,), (1,)), ((), ())),
                              preferred_element_type=jnp.float32)
    xlt = xlt + blt_ref[...]

    xrbt = jax.lax.dot_general(wr_ref[...], xcb_ref[...], (((1,), (1,)), ((), ())),
                               preferred_element_type=jnp.float32)
    xrbt = xrbt + brt_ref[...]

    batch_row = batch_ref[0]
    giota = jax.lax.broadcasted_iota(jnp.int32, (_M, _NB), 0) // _R
    maskt = batch_row == giota

    srow = jax.lax.broadcasted_iota(jnp.int32, (_R, _M), 0)
    scol = jax.lax.broadcasted_iota(jnp.int32, (_R, _M), 1) % _R
    smat = (srow == scol).astype(jnp.float32)

    d_row = jax.lax.broadcasted_iota(jnp.int32, (1, _M), 1)
    ones_row_b = jnp.ones((1, _NB), dtype=jnp.bfloat16)

    _RG = 4
    bdtrow = jax.lax.broadcasted_iota(jnp.int32, (_RG, _RG * _C), 0)
    bdtcol = jax.lax.broadcasted_iota(jnp.int32, (_RG, _RG * _C), 1) // _C
    bdtmask = bdtrow == bdtcol

    for h in range(_H):
        xlt_h = xlt[h * _C:(h + 1) * _C, :]
        xrbt_h = xrbt[h * _C:(h + 1) * _C, :]
        att_row = att_ref[h:h + 1, :]

        a_row = jax.lax.dot_general(att_row, xlt_h, (((1,), (0,)), ((), ())),
                                    preferred_element_type=jnp.float32)
        b_colv = jax.lax.dot_general(xrbt_h, att_row, (((0,), (1,)), ((), ())),
                                     preferred_element_type=jnp.float32)

        attwide = jnp.concatenate([jnp.broadcast_to(att_row, (_RG, _C))] * _RG,
                                  axis=1)
        bdt_h = jnp.where(bdtmask, attwide, 0.0).astype(jnp.bfloat16)
        xlt_hb = xlt_h.astype(jnp.bfloat16)
        xrbt_hb = xrbt_h.astype(jnp.bfloat16)
        gparts = []
        for rg in range(_R // _RG):
            slabs = [jnp.abs(xlt_hb + xrbt_hb[:, r:r + 1])
                     for r in range(rg * _RG, (rg + 1) * _RG)]
            absz = jnp.concatenate(slabs, axis=0)
            gparts.append(jax.lax.dot_general(
                bdt_h, absz, (((1,), (0,)), ((), ())),
                preferred_element_type=jnp.float32))
        g_all = jnp.concatenate(gparts, axis=0)
        l_ht = 0.6 * (a_row + b_colv) + 0.4 * g_all
        e_ht = jnp.exp(l_ht)

        @pl.when(step == 0)
        def _stash():
            e0 = e_ht[:, :_M]
            e_self = jnp.sum(e0 * smat, axis=0, keepdims=True)
            b0 = batch_row[:, :_M]
            corr = (b0 != d_row // _R).astype(jnp.float32)
            corrf_ref[h] = e_self * corr

        e_bigt = jax.lax.dot_general(smat, e_ht, (((0,), (0,)), ((), ())),
                                     preferred_element_type=jnp.float32)
        p_ht = jnp.where(maskt, e_bigt, 0.0).astype(jnp.bfloat16)
        rhs = jnp.concatenate([xlt_hb, ones_row_b], axis=0)
        num = jax.lax.dot_general(rhs, p_ht, (((1,), (1,)), ((), ())),
                                  preferred_element_type=jnp.float32)

        @pl.when(step == 0)
        def _init():
            acc_ref[h] = num

        @pl.when(step != 0)
        def _accum():
            acc_ref[h] += num

    @pl.when(step == 0)
    def _stash_xl():
        xlself_ref[...] = xlt[:, :_M]

    @pl.when(step == _NSTEPS - 1)
    def _finalize():
        tot = jnp.zeros((_C, _M), dtype=jnp.float32)
        for h in range(_H):
            corr = corrf_ref[h]
            num = acc_ref[h][:_C, :] + corr * xlself_ref[h * _C:(h + 1) * _C, :]
            den = acc_ref[h][_C:_C + 1, :] + corr + 1e-16
            tot = tot + num / den
        out_ref[...] = tot * (1.0 / _H) + biast_ref[...]


@functools.partial(jax.jit, static_argnames=("interpret",))
def _run(x, batch, xcent_base, Wl, bl, Wr, br, att, bias_out, interpret=False):
    batch3 = batch.reshape(_NSTEPS, 1, _NB)
    grid = (_NSTEPS,)
    xcent_t = pl.pallas_call(
        _bipart_kernel,
        grid=grid,
        in_specs=[
            pl.BlockSpec((_NB, _C), lambda i: (i, 0)),
            pl.BlockSpec((1, 1, _NB), lambda i: (i, 0, 0)),
            pl.BlockSpec((_F, _C), lambda i: (0, 0)),
            pl.BlockSpec((_F, 1), lambda i: (0, 0)),
            pl.BlockSpec((_R, _C), lambda i: (0, 0)),
            pl.BlockSpec((_F, _C), lambda i: (0, 0)),
            pl.BlockSpec((_F, 1), lambda i: (0, 0)),
            pl.BlockSpec((_H, _C), lambda i: (0, 0)),
            pl.BlockSpec((_C, 1), lambda i: (0, 0)),
        ],
        out_specs=pl.BlockSpec((_C, _M), lambda i: (0, 0)),
        out_shape=jax.ShapeDtypeStruct((_C, _M), jnp.float32),
        scratch_shapes=[
            pltpu.VMEM((_H, _C + 1, _M), jnp.float32),
            pltpu.VMEM((_F, _M), jnp.float32),
            pltpu.VMEM((_H, 1, _M), jnp.float32),
        ],
        compiler_params=pltpu.CompilerParams(
            dimension_semantics=("arbitrary",),
        ),
        interpret=interpret,
    )(x.astype(jnp.bfloat16), batch3, Wl.astype(jnp.bfloat16),
      bl.reshape(_F, 1), xcent_base, Wr, br.reshape(_F, 1),
      att, bias_out.reshape(_C, 1))
    return xcent_t.T


def kernel(x, edge_index, batch, xcent_base, Wl, bl, Wr, br, att, bias_out):
    del edge_index
    xcent = _run(x, batch, xcent_base, Wl, bl, Wr, br, att, bias_out)
    batchcent = jnp.repeat(jnp.arange(_G, dtype=jnp.int32), _R)
    return (xcent, batchcent)


# speedup vs baseline: 1.0004x; 1.0004x over previous
import jax
import jax.numpy as jnp
from jax.experimental import pallas as pl
from jax.experimental.pallas import tpu as pltpu

_N = 10000
_C = 128
_H = 4
_R = 16
_G = 16
_M = 256
_F = _H * _C
_NB = 10000
_NSTEPS = _N // _NB


def _bipart_kernel(x_ref, batch_ref, wl_ref, blt_ref, xcb_ref, wr_ref, brt_ref,
                   att_ref, biast_ref, out_ref, acc_ref, xlself_ref,
                   corrf_ref):
    step = pl.program_id(0)

    x_blk = x_ref[...]
    xlt = jax.lax.dot_general(wl_ref[...], x_blk, (((1,), (1,)), ((), ())),
                              preferred_element_type=jnp.float32)
    xlt = xlt + blt_ref[...]

    xrbt = jax.lax.dot_general(wr_ref[...], xcb_ref[...], (((1,), (1,)), ((), ())),
                               preferred_element_type=jnp.float32)
    xrbt = xrbt + brt_ref[...]

    batch_row = batch_ref[0]
    giota = jax.lax.broadcasted_iota(jnp.int32, (_M, _NB), 0) // _R
    maskt = batch_row == giota

    srow = jax.lax.broadcasted_iota(jnp.int32, (_R, _M), 0)
    scol = jax.lax.broadcasted_iota(jnp.int32, (_R, _M), 1) % _R
    smat = (srow == scol).astype(jnp.float32)

    d_row = jax.lax.broadcasted_iota(jnp.int32, (1, _M), 1)
    ones_row_b = jnp.ones((1, _NB), dtype=jnp.bfloat16)

    _RG = 4
    bdtrow = jax.lax.broadcasted_iota(jnp.int32, (_RG, _RG * _C), 0)
    bdtcol = jax.lax.broadcasted_iota(jnp.int32, (_RG, _RG * _C), 1) // _C
    bdtmask = bdtrow == bdtcol

    for h in range(_H):
        xlt_h = xlt[h * _C:(h + 1) * _C, :]
        xrbt_h = xrbt[h * _C:(h + 1) * _C, :]
        att_row = att_ref[h:h + 1, :]

        a_row = jax.lax.dot_general(att_row, xlt_h, (((1,), (0,)), ((), ())),
                                    preferred_element_type=jnp.float32)
        b_colv = jax.lax.dot_general(xrbt_h, att_row, (((0,), (1,)), ((), ())),
                                     preferred_element_type=jnp.float32)

        attwide = jnp.concatenate([jnp.broadcast_to(att_row, (_RG, _C))] * _RG,
                                  axis=1)
        bdt_h = jnp.where(bdtmask, attwide, 0.0).astype(jnp.bfloat16)
        xlt_hb = xlt_h.astype(jnp.bfloat16)
        xrbt_hb = xrbt_h.astype(jnp.bfloat16)
        gparts = []
        for rg in range(_R // _RG):
            slabs = [jnp.abs(xlt_hb + xrbt_hb[:, r:r + 1])
                     for r in range(rg * _RG, (rg + 1) * _RG)]
            absz = jnp.concatenate(slabs, axis=0)
            gparts.append(jax.lax.dot_general(
                bdt_h, absz, (((1,), (0,)), ((), ())),
                preferred_element_type=jnp.float32))
        g_all = jnp.concatenate(gparts, axis=0)
        l_ht = 0.6 * (a_row + b_colv) + 0.4 * g_all
        e_ht = jnp.exp(l_ht)

        @pl.when(step == 0)
        def _stash():
            e0 = e_ht[:, :_M]
            e_self = jnp.sum(e0 * smat, axis=0, keepdims=True)
            b0 = batch_row[:, :_M]
            corr = (b0 != d_row // _R).astype(jnp.float32)
            corrf_ref[h] = e_self * corr

        e_bigt = jax.lax.dot_general(smat, e_ht, (((0,), (0,)), ((), ())),
                                     preferred_element_type=jnp.float32)
        p_ht = jnp.where(maskt, e_bigt, 0.0).astype(jnp.bfloat16)
        rhs = jnp.concatenate([xlt_hb, ones_row_b], axis=0)
        num = jax.lax.dot_general(rhs, p_ht, (((1,), (1,)), ((), ())),
                                  preferred_element_type=jnp.float32)

        @pl.when(step == 0)
        def _init():
            acc_ref[h] = num

        @pl.when(step != 0)
        def _accum():
            acc_ref[h] += num

    @pl.when(step == 0)
    def _stash_xl():
        xlself_ref[...] = xlt[:, :_M]

    @pl.when(step == _NSTEPS - 1)
    def _finalize():
        tot = jnp.zeros((_C, _M), dtype=jnp.float32)
        for h in range(_H):
            corr = corrf_ref[h]
            num = acc_ref[h][:_C, :] + corr * xlself_ref[h * _C:(h + 1) * _C, :]
            den = acc_ref[h][_C:_C + 1, :] + corr + 1e-16
            tot = tot + num / den
        out_ref[...] = tot * (1.0 / _H) + biast_ref[...]


@jax.jit
def _run(x, batch, xcent_base, Wl, bl, Wr, br, att, bias_out):
    batch3 = batch.reshape(_NSTEPS, 1, _NB)
    grid = (_NSTEPS,)
    xcent_t = pl.pallas_call(
        _bipart_kernel,
        grid=grid,
        in_specs=[
            pl.BlockSpec((_NB, _C), lambda i: (i, 0)),
            pl.BlockSpec((1, 1, _NB), lambda i: (i, 0, 0)),
            pl.BlockSpec((_F, _C), lambda i: (0, 0)),
            pl.BlockSpec((_F, 1), lambda i: (0, 0)),
            pl.BlockSpec((_R, _C), lambda i: (0, 0)),
            pl.BlockSpec((_F, _C), lambda i: (0, 0)),
            pl.BlockSpec((_F, 1), lambda i: (0, 0)),
            pl.BlockSpec((_H, _C), lambda i: (0, 0)),
            pl.BlockSpec((_C, 1), lambda i: (0, 0)),
        ],
        out_specs=pl.BlockSpec((_C, _M), lambda i: (0, 0)),
        out_shape=jax.ShapeDtypeStruct((_C, _M), jnp.float32),
        scratch_shapes=[
            pltpu.VMEM((_H, _C + 1, _M), jnp.float32),
            pltpu.VMEM((_F, _M), jnp.float32),
            pltpu.VMEM((_H, 1, _M), jnp.float32),
        ],
        compiler_params=pltpu.CompilerParams(
            dimension_semantics=("arbitrary",),
        ),
    )(x.astype(jnp.bfloat16), batch3, Wl.astype(jnp.bfloat16),
      bl.reshape(_F, 1), xcent_base, Wr, br.reshape(_F, 1),
      att, bias_out.reshape(_C, 1))
    return xcent_t.T


def kernel(x, edge_index, batch, xcent_base, Wl, bl, Wr, br, att, bias_out):
    del edge_index
    xcent = _run(x, batch, xcent_base, Wl, bl, Wr, br, att, bias_out)
    batchcent = jnp.repeat(jnp.arange(_G, dtype=jnp.int32), _R)
    return (xcent, batchcent)
